# Initial kernel scaffold; baseline (speedup 1.0000x reference)
import functools

import jax
import jax.numpy as jnp
from jax import lax
from jax.experimental import pallas as pl
from jax.experimental.pallas import tpu as pltpu
from jax.experimental.pallas import tpu_sc as plsc

N_NODES = 10000
N_EDGES = 320000
D = 128
NUM_GRAPHS = 64

NC = 2
NS = 16
NW = NC * NS
EPW = N_EDGES // NW
C = 80
NCHUNK = EPW // C
RPS = N_NODES // NS
ZROWS = 125


def _edge_body(x_hbm, ea_hbm, src_hbm, dst_hbm, out_hbm,
               sidx, didx, rows, eab, zbuf, aggr, sem):
    c = lax.axis_index("c")
    s = lax.axis_index("s")

    zero16 = jnp.zeros((16,), jnp.float32)

    def zero_zbuf(i, carry):
        for j in range(8):
            zbuf[i, 16 * j:16 * (j + 1)] = zero16
        return carry
    lax.fori_loop(0, ZROWS, zero_zbuf, 0)

    def zero_aggr(k, carry):
        pltpu.sync_copy(zbuf, aggr.at[pl.ds(s * RPS + k * ZROWS, ZROWS)])
        return carry
    lax.fori_loop(0, RPS // ZROWS, zero_aggr, 0)

    plsc.subcore_barrier()

    base_w = (c * NS + s) * EPW

    def chunk(k, carry):
        b = base_w + k * C
        pltpu.sync_copy(src_hbm.at[pl.ds(b, C)], sidx)
        pltpu.sync_copy(dst_hbm.at[pl.ds(b, C)], didx)
        pltpu.async_copy(x_hbm.at[sidx], rows, sem).wait()
        pltpu.sync_copy(ea_hbm.at[pl.ds(b, C)], eab)

        def row(i, rcarry):
            for j in range(8):
                sl = pl.ds(16 * j, 16)
                rows[i, sl] = jnp.maximum(rows[i, sl] + eab[i, sl], 0.0)
            return rcarry
        lax.fori_loop(0, C, row, 0)

        pltpu.sync_copy(rows, aggr.at[didx], add=True)
        return carry
    lax.fori_loop(0, NCHUNK, chunk, 0)

    plsc.subcore_barrier()

    def writeback(k, carry):
        r0 = s * RPS + k * ZROWS
        pltpu.sync_copy(aggr.at[pl.ds(r0, ZROWS)],
                        out_hbm.at[c, pl.ds(r0, ZROWS)])
        return carry
    lax.fori_loop(0, RPS // ZROWS, writeback, 0)


_edge_phase = functools.partial(
    pl.kernel,
    out_type=jax.ShapeDtypeStruct((NC, N_NODES, D), jnp.float32),
    mesh=plsc.VectorSubcoreMesh(core_axis_name="c", subcore_axis_name="s"),
    scratch_types=[
        pltpu.VMEM((C,), jnp.int32),
        pltpu.VMEM((C,), jnp.int32),
        pltpu.VMEM((C, D), jnp.float32),
        pltpu.VMEM((C, D), jnp.float32),
        pltpu.VMEM((ZROWS, D), jnp.float32),
        pltpu.VMEM_SHARED((N_NODES, D), jnp.float32),
        pltpu.SemaphoreType.DMA,
    ],
)(_edge_body)


def _node_enc_body(x_ref, w_ref, b_ref, out_ref):
    out_ref[...] = (
        jnp.dot(x_ref[...], w_ref[...], preferred_element_type=jnp.float32)
        + b_ref[...])


def _node_enc(x, w, b):
    return pl.pallas_call(
        _node_enc_body,
        out_shape=jax.ShapeDtypeStruct((N_NODES, D), jnp.float32),
    )(x, w, b.reshape(1, D))


EBLK = 8000


def _edge_enc_body(ea_ref, w_ref, b_ref, out_ref):
    out_ref[...] = (
        jnp.dot(ea_ref[...], w_ref[...], preferred_element_type=jnp.float32)
        + b_ref[...])


def _edge_enc(edge_attrs, w, b):
    de = edge_attrs.shape[1]
    return pl.pallas_call(
        _edge_enc_body,
        grid=(N_EDGES // EBLK,),
        in_specs=[
            pl.BlockSpec((EBLK, de), lambda i: (i, 0)),
            pl.BlockSpec((de, D), lambda i: (0, 0)),
            pl.BlockSpec((1, D), lambda i: (0, 0)),
        ],
        out_specs=pl.BlockSpec((EBLK, D), lambda i: (i, 0)),
        out_shape=jax.ShapeDtypeStruct((N_EDGES, D), jnp.float32),
    )(edge_attrs, w, b.reshape(1, D))


def _dense_body(aggr2_ref, x_ref, w1_ref, b1_ref, g1_ref, bb1_ref,
                w2_ref, b2_ref, lg_ref, lb_ref, out_ref):
    x = x_ref[...]
    h0 = x + aggr2_ref[0] + aggr2_ref[1]
    h = jnp.dot(h0, w1_ref[...], preferred_element_type=jnp.float32) + b1_ref[...]
    mu = jnp.mean(h, axis=0, keepdims=True)
    var = jnp.mean((h - mu) ** 2, axis=0, keepdims=True)
    hb = (h - mu) * jax.lax.rsqrt(var + 1e-5) * g1_ref[...] + bb1_ref[...]
    hb = jnp.maximum(hb, 0.0)
    ge = jnp.dot(hb, w2_ref[...], preferred_element_type=jnp.float32) + b2_ref[...]
    t = jnp.maximum(x + jnp.maximum(ge, 0.0), 0.0)
    m = jnp.mean(t)
    v = jnp.mean((t - m) ** 2)
    out_ref[...] = (t - m) * jax.lax.rsqrt(v + 1e-5) * lg_ref[...] + lb_ref[...]


def _dense_layer(aggr2, x, conv, ln):
    h1 = conv['W1'].shape[1]
    return pl.pallas_call(
        _dense_body,
        out_shape=jax.ShapeDtypeStruct((N_NODES, D), jnp.float32),
    )(aggr2, x, conv['W1'], conv['b1'].reshape(1, h1),
      conv['bn_g'].reshape(1, h1), conv['bn_b'].reshape(1, h1),
      conv['W2'], conv['b2'].reshape(1, D),
      ln['g'].reshape(1, D), ln['b'].reshape(1, D))


def _pool_body(x_ref, batch_ref, gw1_ref, gb1_ref, gw2_ref, gb2_ref,
               cw1_ref, cb1_ref, cw2_ref, cb2_ref, cw3_ref, cb3_ref,
               out_ref):
    x = x_ref[...]
    batch = batch_ref[...]
    gid = jax.lax.broadcasted_iota(jnp.int32, (N_NODES, NUM_GRAPHS), 1)
    eq = batch == gid
    onehot = jnp.where(eq, 1.0, 0.0)

    gate = (jnp.dot(jnp.maximum(
        jnp.dot(x, gw1_ref[...], preferred_element_type=jnp.float32)
        + gb1_ref[...], 0.0), gw2_ref[...],
        preferred_element_type=jnp.float32) + gb2_ref[...])

    masked = jnp.where(eq, gate, -1e30)
    gmax = jnp.max(masked, axis=0, keepdims=True)
    gm_node = jnp.sum(onehot * gmax, axis=1, keepdims=True)
    e = jnp.exp(gate - gm_node)
    den = jax.lax.dot_general(
        onehot, e, (((0,), (0,)), ((), ())),
        preferred_element_type=jnp.float32)
    den_node = jnp.dot(onehot, den, preferred_element_type=jnp.float32)
    alpha = e / den_node
    att = jax.lax.dot_general(
        onehot, alpha * x, (((0,), (0,)), ((), ())),
        preferred_element_type=jnp.float32)
    addp = jax.lax.dot_general(
        onehot, x, (((0,), (0,)), ((), ())),
        preferred_element_type=jnp.float32)

    g = jnp.concatenate([att, addp], axis=1)
    h = jnp.maximum(
        jnp.dot(g, cw1_ref[...], preferred_element_type=jnp.float32)
        + cb1_ref[...], 0.0)
    h = jnp.maximum(
        jnp.dot(h, cw2_ref[...], preferred_element_type=jnp.float32)
        + cb2_ref[...], 0.0)
    out_ref[...] = (
        jnp.dot(h, cw3_ref[...], preferred_element_type=jnp.float32)
        + cb3_ref[...])


def _pool(x, batch, gp, cp):
    return pl.pallas_call(
        _pool_body,
        out_shape=jax.ShapeDtypeStruct((NUM_GRAPHS, 1), jnp.float32),
    )(x, batch.reshape(N_NODES, 1),
      gp['W1'], gp['b1'].reshape(1, -1), gp['W2'], gp['b2'].reshape(1, -1),
      cp['W1'], cp['b1'].reshape(1, -1), cp['W2'], cp['b2'].reshape(1, -1),
      cp['W3'], cp['b3'].reshape(1, -1))


def kernel(x, edge_index, edge_attrs, batch, params):
    src = edge_index[0]
    dst = edge_index[1]
    xi = _node_enc(x, params['node_enc']['W'], params['node_enc']['b'])
    ea = _edge_enc(edge_attrs, params['edge_enc']['W'], params['edge_enc']['b'])
    for i in (1, 2, 3, 4):
        aggr2 = _edge_phase(xi, ea, src, dst)
        xi = _dense_layer(aggr2, xi, params['conv%d' % i], params['ln%d' % i])
    return _pool(xi, batch, params['gate'], params['cls'])


# speedup vs baseline: 2.8493x; 2.8493x over previous
import functools

import jax
import jax.numpy as jnp
from jax import lax
from jax.experimental import pallas as pl
from jax.experimental.pallas import tpu as pltpu
from jax.experimental.pallas import tpu_sc as plsc

N_NODES = 10000
N_EDGES = 320000
D = 128
NUM_GRAPHS = 64

NC = 2
NS = 16
NW = NC * NS
EPW = N_EDGES // NW
C = 80
NCHUNK = EPW // C
ZROWS = 200
NWB = N_NODES // ZROWS


def _edge_body(x_hbm, ea_hbm, src_hbm, dst_hbm, out_hbm,
               sidx, didx, rows, eab, zbuf, aggr, sem):
    c = lax.axis_index("c")
    s = lax.axis_index("s")

    zero16 = jnp.zeros((16,), jnp.float32)

    def zero_zbuf(i, carry):
        for j in range(8):
            zbuf[i, 16 * j:16 * (j + 1)] = zero16
        return carry
    lax.fori_loop(0, ZROWS, zero_zbuf, 0)

    def zero_aggr(k, carry):
        idx = k * NS + s
        @pl.when(idx < NWB)
        def _():
            pltpu.sync_copy(zbuf, aggr.at[pl.ds(idx * ZROWS, ZROWS)])
        return carry
    lax.fori_loop(0, (NWB + NS - 1) // NS, zero_aggr, 0)

    plsc.subcore_barrier()

    base_w = (c * NS + s) * EPW

    def chunk(k, carry):
        b = base_w + k * C
        pltpu.sync_copy(src_hbm.at[pl.ds(b, C)], sidx)
        pltpu.sync_copy(dst_hbm.at[pl.ds(b, C)], didx)
        pltpu.async_copy(x_hbm.at[sidx], rows, sem).wait()
        pltpu.sync_copy(ea_hbm.at[pl.ds(b, C)], eab)

        def row(i, rcarry):
            for j in range(8):
                sl = pl.ds(16 * j, 16)
                rows[i, sl] = jnp.maximum(rows[i, sl] + eab[i, sl], 0.0)
            return rcarry
        lax.fori_loop(0, C, row, 0)

        pltpu.sync_copy(rows, aggr.at[didx], add=True)
        return carry
    lax.fori_loop(0, NCHUNK, chunk, 0)

    plsc.subcore_barrier()

    def writeback(k, carry):
        idx = k * NS + s
        @pl.when(idx < NWB)
        def _():
            r0 = idx * ZROWS
            pltpu.sync_copy(aggr.at[pl.ds(r0, ZROWS)],
                            out_hbm.at[c, pl.ds(r0, ZROWS)])
        return carry
    lax.fori_loop(0, (NWB + NS - 1) // NS, writeback, 0)


@functools.cache
def _edge_phase_kernel():
    return pl.kernel(
        _edge_body,
        out_type=jax.ShapeDtypeStruct((NC, N_NODES, D), jnp.float32),
        mesh=plsc.VectorSubcoreMesh(core_axis_name="c", subcore_axis_name="s"),
        scratch_types=[
            pltpu.VMEM((C,), jnp.int32),
            pltpu.VMEM((C,), jnp.int32),
            pltpu.VMEM((C, D), jnp.float32),
            pltpu.VMEM((C, D), jnp.float32),
            pltpu.VMEM((ZROWS, D), jnp.float32),
            pltpu.VMEM_SHARED((N_NODES, D), jnp.float32),
            pltpu.SemaphoreType.DMA,
        ],
    )


def _edge_phase(x, ea, src, dst):
    return _edge_phase_kernel()(x, ea, src, dst)


def _node_enc_body(x_ref, w_ref, b_ref, out_ref):
    out_ref[...] = (
        jnp.dot(x_ref[...], w_ref[...], preferred_element_type=jnp.float32)
        + b_ref[...])


def _node_enc(x, w, b):
    return pl.pallas_call(
        _node_enc_body,
        out_shape=jax.ShapeDtypeStruct((N_NODES, D), jnp.float32),
    )(x, w, b.reshape(1, D))


EBLK = 8000


def _edge_enc_body(ea_ref, w_ref, b_ref, out_ref):
    out_ref[...] = (
        jnp.dot(ea_ref[...], w_ref[...], preferred_element_type=jnp.float32)
        + b_ref[...])


def _edge_enc(edge_attrs, w, b):
    de = edge_attrs.shape[1]
    return pl.pallas_call(
        _edge_enc_body,
        grid=(N_EDGES // EBLK,),
        in_specs=[
            pl.BlockSpec((EBLK, de), lambda i: (i, 0)),
            pl.BlockSpec((de, D), lambda i: (0, 0)),
            pl.BlockSpec((1, D), lambda i: (0, 0)),
        ],
        out_specs=pl.BlockSpec((EBLK, D), lambda i: (i, 0)),
        out_shape=jax.ShapeDtypeStruct((N_EDGES, D), jnp.float32),
    )(edge_attrs, w, b.reshape(1, D))


def _dense_body(aggr2_ref, x_ref, w1_ref, b1_ref, g1_ref, bb1_ref,
                w2_ref, b2_ref, lg_ref, lb_ref, out_ref):
    x = x_ref[...]
    h0 = x + aggr2_ref[0] + aggr2_ref[1]
    h = jnp.dot(h0, w1_ref[...], preferred_element_type=jnp.float32) + b1_ref[...]
    mu = jnp.mean(h, axis=0, keepdims=True)
    var = jnp.mean((h - mu) ** 2, axis=0, keepdims=True)
    hb = (h - mu) * jax.lax.rsqrt(var + 1e-5) * g1_ref[...] + bb1_ref[...]
    hb = jnp.maximum(hb, 0.0)
    ge = jnp.dot(hb, w2_ref[...], preferred_element_type=jnp.float32) + b2_ref[...]
    t = jnp.maximum(x + jnp.maximum(ge, 0.0), 0.0)
    m = jnp.mean(t)
    v = jnp.mean((t - m) ** 2)
    out_ref[...] = (t - m) * jax.lax.rsqrt(v + 1e-5) * lg_ref[...] + lb_ref[...]


def _dense_layer(aggr2, x, conv, ln):
    h1 = conv['W1'].shape[1]
    return pl.pallas_call(
        _dense_body,
        out_shape=jax.ShapeDtypeStruct((N_NODES, D), jnp.float32),
    )(aggr2, x, conv['W1'], conv['b1'].reshape(1, h1),
      conv['bn_g'].reshape(1, h1), conv['bn_b'].reshape(1, h1),
      conv['W2'], conv['b2'].reshape(1, D),
      ln['g'].reshape(1, D), ln['b'].reshape(1, D))


def _pool_body(x_ref, batch_ref, gw1_ref, gb1_ref, gw2_ref, gb2_ref,
               cw1_ref, cb1_ref, cw2_ref, cb2_ref, cw3_ref, cb3_ref,
               out_ref):
    x = x_ref[...]
    batch = batch_ref[...]
    gid = jax.lax.broadcasted_iota(jnp.int32, (N_NODES, NUM_GRAPHS), 1)
    eq = batch == gid
    onehot = jnp.where(eq, 1.0, 0.0)

    gate = (jnp.dot(jnp.maximum(
        jnp.dot(x, gw1_ref[...], preferred_element_type=jnp.float32)
        + gb1_ref[...], 0.0), gw2_ref[...],
        preferred_element_type=jnp.float32) + gb2_ref[...])

    masked = jnp.where(eq, gate, -1e30)
    gmax = jnp.max(masked, axis=0, keepdims=True)
    gm_node = jnp.sum(onehot * gmax, axis=1, keepdims=True)
    e = jnp.exp(gate - gm_node)
    den = jax.lax.dot_general(
        onehot, e, (((0,), (0,)), ((), ())),
        preferred_element_type=jnp.float32)
    den_node = jnp.dot(onehot, den, preferred_element_type=jnp.float32)
    alpha = e / den_node
    att = jax.lax.dot_general(
        onehot, alpha * x, (((0,), (0,)), ((), ())),
        preferred_element_type=jnp.float32)
    addp = jax.lax.dot_general(
        onehot, x, (((0,), (0,)), ((), ())),
        preferred_element_type=jnp.float32)

    g = jnp.concatenate([att, addp], axis=1)
    h = jnp.maximum(
        jnp.dot(g, cw1_ref[...], preferred_element_type=jnp.float32)
        + cb1_ref[...], 0.0)
    h = jnp.maximum(
        jnp.dot(h, cw2_ref[...], preferred_element_type=jnp.float32)
        + cb2_ref[...], 0.0)
    out_ref[...] = (
        jnp.dot(h, cw3_ref[...], preferred_element_type=jnp.float32)
        + cb3_ref[...])


def _pool(x, batch, gp, cp):
    return pl.pallas_call(
        _pool_body,
        out_shape=jax.ShapeDtypeStruct((NUM_GRAPHS, 1), jnp.float32),
    )(x, batch.reshape(N_NODES, 1),
      gp['W1'], gp['b1'].reshape(1, -1), gp['W2'], gp['b2'].reshape(1, -1),
      cp['W1'], cp['b1'].reshape(1, -1), cp['W2'], cp['b2'].reshape(1, -1),
      cp['W3'], cp['b3'].reshape(1, -1))


def kernel(x, edge_index, edge_attrs, batch, params):
    src = edge_index[0]
    dst = edge_index[1]
    xi = _node_enc(x, params['node_enc']['W'], params['node_enc']['b'])
    ea = _edge_enc(edge_attrs, params['edge_enc']['W'], params['edge_enc']['b'])
    for i in (1, 2, 3, 4):
        aggr2 = _edge_phase(xi, ea, src, dst)
        xi = _dense_layer(aggr2, xi, params['conv%d' % i], params['ln%d' % i])
    return _pool(xi, batch, params['gate'], params['cls'])


# speedup vs baseline: 5.6338x; 1.9773x over previous
import functools

import jax
import jax.numpy as jnp
from jax import lax
from jax.experimental import pallas as pl
from jax.experimental.pallas import tpu as pltpu
from jax.experimental.pallas import tpu_sc as plsc

N_NODES = 10000
N_EDGES = 320000
D = 128
NUM_GRAPHS = 64

NC = 2
NS = 16
NW = NC * NS
EPW = N_EDGES // NW
C = 80
NCHUNK = EPW // C
ZROWS = 40
NWB = N_NODES // ZROWS


def _edge_body(x_hbm, ea_hbm, src_hbm, dst_hbm, out_hbm,
               sidx, didx, rows, eab, zbuf, aggr,
               semi, semg0, semg1, sems0, sems1):
    c = lax.axis_index("c")
    s = lax.axis_index("s")

    zero16 = jnp.zeros((16,), jnp.float32)

    def zero_zbuf(i, carry):
        for j in range(8):
            zbuf[i, 16 * j:16 * (j + 1)] = zero16
        return carry
    lax.fori_loop(0, ZROWS, zero_zbuf, 0)

    def zero_aggr(k, carry):
        idx = k * NS + s
        @pl.when(idx < NWB)
        def _():
            pltpu.sync_copy(zbuf, aggr.at[pl.ds(idx * ZROWS, ZROWS)])
        return carry
    lax.fori_loop(0, (NWB + NS - 1) // NS, zero_aggr, 0)

    plsc.subcore_barrier()

    base_w = (c * NS + s) * EPW
    semg = (semg0, semg1)
    sems = (sems0, sems1)

    def fire_idx(k, b):
        e0 = base_w + k * C
        pltpu.async_copy(src_hbm.at[pl.ds(e0, C)], sidx.at[b], semi)
        pltpu.async_copy(dst_hbm.at[pl.ds(e0, C)], didx.at[b], semi)

    def wait_idx(b):
        pltpu.make_async_copy(src_hbm.at[pl.ds(0, C)], sidx.at[b], semi).wait()
        pltpu.make_async_copy(dst_hbm.at[pl.ds(0, C)], didx.at[b], semi).wait()

    def fire_gather(k, b):
        e0 = base_w + k * C
        pltpu.async_copy(x_hbm.at[sidx.at[b]], rows.at[b], semg[b])
        pltpu.async_copy(ea_hbm.at[pl.ds(e0, C)], eab.at[b], semg[b])

    def wait_gather(b):
        pltpu.make_async_copy(x_hbm.at[sidx.at[b]], rows.at[b], semg[b]).wait()
        pltpu.make_async_copy(ea_hbm.at[pl.ds(0, C)], eab.at[b], semg[b]).wait()

    def fire_scatter(b):
        pltpu.async_copy(rows.at[b], aggr.at[didx.at[b]], sems[b], add=True)

    def wait_scatter(b):
        pltpu.make_async_copy(rows.at[b], aggr.at[didx.at[b]], sems[b]).wait()

    fire_idx(0, 0)
    wait_idx(0)
    fire_gather(0, 0)

    def half(k, b):
        nb = 1 - b

        @pl.when(k + 1 < NCHUNK)
        def _():
            @pl.when(k >= 1)
            def _():
                wait_scatter(nb)
            fire_idx(k + 1, nb)

        @pl.when(k < NCHUNK)
        def _():
            wait_gather(b)

        @pl.when(k + 1 < NCHUNK)
        def _():
            wait_idx(nb)
            fire_gather(k + 1, nb)

        @pl.when(k < NCHUNK)
        def _():
            @plsc.parallel_loop(0, C, 1, unroll=2)
            def row(i):
                for j in range(8):
                    sl = pl.ds(16 * j, 16)
                    rows[b, i, sl] = jnp.maximum(
                        rows[b, i, sl] + eab[b, i, sl], 0.0)
            fire_scatter(b)

    def pair(k2, carry):
        half(2 * k2, 0)
        half(2 * k2 + 1, 1)
        return carry
    lax.fori_loop(0, (NCHUNK + 2) // 2, pair, 0)

    wait_scatter(1)
    wait_scatter(0)

    plsc.subcore_barrier()

    def writeback(k, carry):
        idx = k * NS + s
        @pl.when(idx < NWB)
        def _():
            r0 = idx * ZROWS
            pltpu.sync_copy(aggr.at[pl.ds(r0, ZROWS)],
                            out_hbm.at[c, pl.ds(r0, ZROWS)])
        return carry
    lax.fori_loop(0, (NWB + NS - 1) // NS, writeback, 0)


@functools.cache
def _edge_phase_kernel():
    return pl.kernel(
        _edge_body,
        out_type=jax.ShapeDtypeStruct((NC, N_NODES, D), jnp.float32),
        mesh=plsc.VectorSubcoreMesh(core_axis_name="c", subcore_axis_name="s"),
        scratch_types=[
            pltpu.VMEM((2, C), jnp.int32),
            pltpu.VMEM((2, C), jnp.int32),
            pltpu.VMEM((2, C, D), jnp.float32),
            pltpu.VMEM((2, C, D), jnp.float32),
            pltpu.VMEM((ZROWS, D), jnp.float32),
            pltpu.VMEM_SHARED((N_NODES, D), jnp.float32),
            pltpu.SemaphoreType.DMA,
            pltpu.SemaphoreType.DMA,
            pltpu.SemaphoreType.DMA,
            pltpu.SemaphoreType.DMA,
            pltpu.SemaphoreType.DMA,
        ],
    )


def _edge_phase(x, ea, src, dst):
    return _edge_phase_kernel()(x, ea, src, dst)


def _node_enc_body(x_ref, w_ref, b_ref, out_ref):
    out_ref[...] = (
        jnp.dot(x_ref[...], w_ref[...], preferred_element_type=jnp.float32)
        + b_ref[...])


def _node_enc(x, w, b):
    return pl.pallas_call(
        _node_enc_body,
        out_shape=jax.ShapeDtypeStruct((N_NODES, D), jnp.float32),
    )(x, w, b.reshape(1, D))


EBLK = 8000


def _edge_enc_body(ea_ref, w_ref, b_ref, out_ref):
    out_ref[...] = (
        jnp.dot(ea_ref[...], w_ref[...], preferred_element_type=jnp.float32)
        + b_ref[...])


def _edge_enc(edge_attrs, w, b):
    de = edge_attrs.shape[1]
    return pl.pallas_call(
        _edge_enc_body,
        grid=(N_EDGES // EBLK,),
        in_specs=[
            pl.BlockSpec((EBLK, de), lambda i: (i, 0)),
            pl.BlockSpec((de, D), lambda i: (0, 0)),
            pl.BlockSpec((1, D), lambda i: (0, 0)),
        ],
        out_specs=pl.BlockSpec((EBLK, D), lambda i: (i, 0)),
        out_shape=jax.ShapeDtypeStruct((N_EDGES, D), jnp.float32),
    )(edge_attrs, w, b.reshape(1, D))


def _dense_body(aggr2_ref, x_ref, w1_ref, b1_ref, g1_ref, bb1_ref,
                w2_ref, b2_ref, lg_ref, lb_ref, out_ref):
    x = x_ref[...]
    h0 = x + aggr2_ref[0] + aggr2_ref[1]
    h = jnp.dot(h0, w1_ref[...], preferred_element_type=jnp.float32) + b1_ref[...]
    mu = jnp.mean(h, axis=0, keepdims=True)
    var = jnp.mean((h - mu) ** 2, axis=0, keepdims=True)
    hb = (h - mu) * jax.lax.rsqrt(var + 1e-5) * g1_ref[...] + bb1_ref[...]
    hb = jnp.maximum(hb, 0.0)
    ge = jnp.dot(hb, w2_ref[...], preferred_element_type=jnp.float32) + b2_ref[...]
    t = jnp.maximum(x + jnp.maximum(ge, 0.0), 0.0)
    m = jnp.mean(t)
    v = jnp.mean((t - m) ** 2)
    out_ref[...] = (t - m) * jax.lax.rsqrt(v + 1e-5) * lg_ref[...] + lb_ref[...]


def _dense_layer(aggr2, x, conv, ln):
    h1 = conv['W1'].shape[1]
    return pl.pallas_call(
        _dense_body,
        out_shape=jax.ShapeDtypeStruct((N_NODES, D), jnp.float32),
    )(aggr2, x, conv['W1'], conv['b1'].reshape(1, h1),
      conv['bn_g'].reshape(1, h1), conv['bn_b'].reshape(1, h1),
      conv['W2'], conv['b2'].reshape(1, D),
      ln['g'].reshape(1, D), ln['b'].reshape(1, D))


def _pool_body(x_ref, batch_ref, gw1_ref, gb1_ref, gw2_ref, gb2_ref,
               cw1_ref, cb1_ref, cw2_ref, cb2_ref, cw3_ref, cb3_ref,
               out_ref):
    x = x_ref[...]
    batch = batch_ref[...]
    gid = jax.lax.broadcasted_iota(jnp.int32, (N_NODES, NUM_GRAPHS), 1)
    eq = batch == gid
    onehot = jnp.where(eq, 1.0, 0.0)

    gate = (jnp.dot(jnp.maximum(
        jnp.dot(x, gw1_ref[...], preferred_element_type=jnp.float32)
        + gb1_ref[...], 0.0), gw2_ref[...],
        preferred_element_type=jnp.float32) + gb2_ref[...])

    masked = jnp.where(eq, gate, -1e30)
    gmax = jnp.max(masked, axis=0, keepdims=True)
    gm_node = jnp.sum(onehot * gmax, axis=1, keepdims=True)
    e = jnp.exp(gate - gm_node)
    den = jax.lax.dot_general(
        onehot, e, (((0,), (0,)), ((), ())),
        preferred_element_type=jnp.float32)
    den_node = jnp.dot(onehot, den, preferred_element_type=jnp.float32)
    alpha = e / den_node
    att = jax.lax.dot_general(
        onehot, alpha * x, (((0,), (0,)), ((), ())),
        preferred_element_type=jnp.float32)
    addp = jax.lax.dot_general(
        onehot, x, (((0,), (0,)), ((), ())),
        preferred_element_type=jnp.float32)

    g = jnp.concatenate([att, addp], axis=1)
    h = jnp.maximum(
        jnp.dot(g, cw1_ref[...], preferred_element_type=jnp.float32)
        + cb1_ref[...], 0.0)
    h = jnp.maximum(
        jnp.dot(h, cw2_ref[...], preferred_element_type=jnp.float32)
        + cb2_ref[...], 0.0)
    out_ref[...] = (
        jnp.dot(h, cw3_ref[...], preferred_element_type=jnp.float32)
        + cb3_ref[...])


def _pool(x, batch, gp, cp):
    return pl.pallas_call(
        _pool_body,
        out_shape=jax.ShapeDtypeStruct((NUM_GRAPHS, 1), jnp.float32),
    )(x, batch.reshape(N_NODES, 1),
      gp['W1'], gp['b1'].reshape(1, -1), gp['W2'], gp['b2'].reshape(1, -1),
      cp['W1'], cp['b1'].reshape(1, -1), cp['W2'], cp['b2'].reshape(1, -1),
      cp['W3'], cp['b3'].reshape(1, -1))


def kernel(x, edge_index, edge_attrs, batch, params):
    src = edge_index[0]
    dst = edge_index[1]
    xi = _node_enc(x, params['node_enc']['W'], params['node_enc']['b'])
    ea = _edge_enc(edge_attrs, params['edge_enc']['W'], params['edge_enc']['b'])
    for i in (1, 2, 3, 4):
        aggr2 = _edge_phase(xi, ea, src, dst)
        xi = _dense_layer(aggr2, xi, params['conv%d' % i], params['ln%d' % i])
    return _pool(xi, batch, params['gate'], params['cls'])


# speedup vs baseline: 5.9115x; 1.0493x over previous
import functools

import jax
import jax.numpy as jnp
from jax import lax
from jax.experimental import pallas as pl
from jax.experimental.pallas import tpu as pltpu
from jax.experimental.pallas import tpu_sc as plsc

N_NODES = 10000
N_EDGES = 320000
D = 128
NUM_GRAPHS = 64

NC = 2
NS = 16
NW = NC * NS
EPW = N_EDGES // NW
C = 80
NCHUNK = EPW // C
ZROWS = 40
NWB = N_NODES // ZROWS


def _edge_body(x_hbm, ea_hbm, src_hbm, dst_hbm, out_hbm,
               sidx, didx, rows, eab, zbuf, aggr,
               semi0, semi1, semi2, semi3, semg0, semg1, sems0, sems1):
    c = lax.axis_index("c")
    s = lax.axis_index("s")

    zero16 = jnp.zeros((16,), jnp.float32)

    def zero_zbuf(i, carry):
        for j in range(8):
            zbuf[i, 16 * j:16 * (j + 1)] = zero16
        return carry
    lax.fori_loop(0, ZROWS, zero_zbuf, 0)

    def zero_aggr(k, carry):
        idx = k * NS + s
        @pl.when(idx < NWB)
        def _():
            pltpu.sync_copy(zbuf, aggr.at[pl.ds(idx * ZROWS, ZROWS)])
        return carry
    lax.fori_loop(0, (NWB + NS - 1) // NS, zero_aggr, 0)

    plsc.subcore_barrier()

    base_w = (c * NS + s) * EPW
    semi = (semi0, semi1, semi2, semi3)
    semg = (semg0, semg1)
    sems = (sems0, sems1)

    def fire_idx(k, b4):
        e0 = base_w + k * C
        pltpu.async_copy(src_hbm.at[pl.ds(e0, C)], sidx.at[b4], semi[b4])
        pltpu.async_copy(dst_hbm.at[pl.ds(e0, C)], didx.at[b4], semi[b4])

    def wait_idx(b4):
        pltpu.make_async_copy(
            src_hbm.at[pl.ds(0, C)], sidx.at[b4], semi[b4]).wait()
        pltpu.make_async_copy(
            dst_hbm.at[pl.ds(0, C)], didx.at[b4], semi[b4]).wait()

    def fire_gather(k, b2, b4):
        e0 = base_w + k * C
        pltpu.async_copy(x_hbm.at[sidx.at[b4]], rows.at[b2], semg[b2])
        pltpu.async_copy(ea_hbm.at[pl.ds(e0, C)], eab.at[b2], semg[b2])

    def wait_gather(b2, b4):
        pltpu.make_async_copy(
            x_hbm.at[sidx.at[b4]], rows.at[b2], semg[b2]).wait()
        pltpu.make_async_copy(
            ea_hbm.at[pl.ds(0, C)], eab.at[b2], semg[b2]).wait()

    def fire_scatter(b2, b4):
        pltpu.async_copy(rows.at[b2], aggr.at[didx.at[b4]], sems[b2], add=True)

    def wait_scatter(b2, b4):
        pltpu.make_async_copy(
            rows.at[b2], aggr.at[didx.at[b4]], sems[b2]).wait()

    fire_idx(0, 0)
    fire_idx(1, 1)
    wait_idx(0)
    fire_gather(0, 0, 0)

    def half(k, b2, b4):
        nb2 = 1 - b2

        @pl.when((k >= 1) & (k + 1 < NCHUNK))
        def _():
            wait_scatter(nb2, (b4 + 3) % 4)

        @pl.when(k + 2 < NCHUNK)
        def _():
            fire_idx(k + 2, (b4 + 2) % 4)

        @pl.when(k < NCHUNK)
        def _():
            wait_gather(b2, b4)

        @pl.when(k + 1 < NCHUNK)
        def _():
            wait_idx((b4 + 1) % 4)
            fire_gather(k + 1, nb2, (b4 + 1) % 4)

        @pl.when(k < NCHUNK)
        def _():
            @plsc.parallel_loop(0, C, 1, unroll=2)
            def row(i):
                for j in range(8):
                    sl = pl.ds(16 * j, 16)
                    rows[b2, i, sl] = jnp.maximum(
                        rows[b2, i, sl] + eab[b2, i, sl], 0.0)
            fire_scatter(b2, b4)

    def quad(k4, carry):
        for q in range(4):
            half(4 * k4 + q, q % 2, q)
        return carry
    lax.fori_loop(0, (NCHUNK + 3) // 4, quad, 0)

    wait_scatter((NCHUNK - 1) % 2, (NCHUNK - 1) % 4)
    wait_scatter((NCHUNK - 2) % 2, (NCHUNK - 2) % 4)

    plsc.subcore_barrier()

    def writeback(k, carry):
        idx = k * NS + s
        @pl.when(idx < NWB)
        def _():
            r0 = idx * ZROWS
            pltpu.sync_copy(aggr.at[pl.ds(r0, ZROWS)],
                            out_hbm.at[c, pl.ds(r0, ZROWS)])
        return carry
    lax.fori_loop(0, (NWB + NS - 1) // NS, writeback, 0)


@functools.cache
def _edge_phase_kernel():
    return pl.kernel(
        _edge_body,
        out_type=jax.ShapeDtypeStruct((NC, N_NODES, D), jnp.float32),
        mesh=plsc.VectorSubcoreMesh(core_axis_name="c", subcore_axis_name="s"),
        scratch_types=[
            pltpu.VMEM((4, C), jnp.int32),
            pltpu.VMEM((4, C), jnp.int32),
            pltpu.VMEM((2, C, D), jnp.float32),
            pltpu.VMEM((2, C, D), jnp.float32),
            pltpu.VMEM((ZROWS, D), jnp.float32),
            pltpu.VMEM_SHARED((N_NODES, D), jnp.float32),
        ] + [pltpu.SemaphoreType.DMA] * 8,
    )


def _edge_phase(x, ea, src, dst):
    return _edge_phase_kernel()(x, ea, src, dst)


def _node_enc_body(x_ref, w_ref, b_ref, out_ref):
    out_ref[...] = (
        jnp.dot(x_ref[...], w_ref[...], preferred_element_type=jnp.float32)
        + b_ref[...])


def _node_enc(x, w, b):
    return pl.pallas_call(
        _node_enc_body,
        out_shape=jax.ShapeDtypeStruct((N_NODES, D), jnp.float32),
    )(x, w, b.reshape(1, D))


EBLK = 8000


def _edge_enc_body(ea_ref, w_ref, b_ref, out_ref):
    out_ref[...] = (
        jnp.dot(ea_ref[...], w_ref[...], preferred_element_type=jnp.float32)
        + b_ref[...])


def _edge_enc(edge_attrs, w, b):
    de = edge_attrs.shape[1]
    return pl.pallas_call(
        _edge_enc_body,
        grid=(N_EDGES // EBLK,),
        in_specs=[
            pl.BlockSpec((EBLK, de), lambda i: (i, 0)),
            pl.BlockSpec((de, D), lambda i: (0, 0)),
            pl.BlockSpec((1, D), lambda i: (0, 0)),
        ],
        out_specs=pl.BlockSpec((EBLK, D), lambda i: (i, 0)),
        out_shape=jax.ShapeDtypeStruct((N_EDGES, D), jnp.float32),
    )(edge_attrs, w, b.reshape(1, D))


def _dense_body(aggr2_ref, x_ref, w1_ref, b1_ref, g1_ref, bb1_ref,
                w2_ref, b2_ref, lg_ref, lb_ref, out_ref):
    x = x_ref[...]
    h0 = x + aggr2_ref[0] + aggr2_ref[1]
    h = jnp.dot(h0, w1_ref[...], preferred_element_type=jnp.float32) + b1_ref[...]
    mu = jnp.mean(h, axis=0, keepdims=True)
    var = jnp.mean((h - mu) ** 2, axis=0, keepdims=True)
    hb = (h - mu) * jax.lax.rsqrt(var + 1e-5) * g1_ref[...] + bb1_ref[...]
    hb = jnp.maximum(hb, 0.0)
    ge = jnp.dot(hb, w2_ref[...], preferred_element_type=jnp.float32) + b2_ref[...]
    t = jnp.maximum(x + jnp.maximum(ge, 0.0), 0.0)
    m = jnp.mean(t)
    v = jnp.mean((t - m) ** 2)
    out_ref[...] = (t - m) * jax.lax.rsqrt(v + 1e-5) * lg_ref[...] + lb_ref[...]


def _dense_layer(aggr2, x, conv, ln):
    h1 = conv['W1'].shape[1]
    return pl.pallas_call(
        _dense_body,
        out_shape=jax.ShapeDtypeStruct((N_NODES, D), jnp.float32),
    )(aggr2, x, conv['W1'], conv['b1'].reshape(1, h1),
      conv['bn_g'].reshape(1, h1), conv['bn_b'].reshape(1, h1),
      conv['W2'], conv['b2'].reshape(1, D),
      ln['g'].reshape(1, D), ln['b'].reshape(1, D))


def _pool_body(x_ref, batch_ref, gw1_ref, gb1_ref, gw2_ref, gb2_ref,
               cw1_ref, cb1_ref, cw2_ref, cb2_ref, cw3_ref, cb3_ref,
               out_ref):
    x = x_ref[...]
    batch = batch_ref[...]
    gid = jax.lax.broadcasted_iota(jnp.int32, (N_NODES, NUM_GRAPHS), 1)
    eq = batch == gid
    onehot = jnp.where(eq, 1.0, 0.0)

    gate = (jnp.dot(jnp.maximum(
        jnp.dot(x, gw1_ref[...], preferred_element_type=jnp.float32)
        + gb1_ref[...], 0.0), gw2_ref[...],
        preferred_element_type=jnp.float32) + gb2_ref[...])

    masked = jnp.where(eq, gate, -1e30)
    gmax = jnp.max(masked, axis=0, keepdims=True)
    gm_node = jnp.sum(onehot * gmax, axis=1, keepdims=True)
    e = jnp.exp(gate - gm_node)
    den = jax.lax.dot_general(
        onehot, e, (((0,), (0,)), ((), ())),
        preferred_element_type=jnp.float32)
    den_node = jnp.dot(onehot, den, preferred_element_type=jnp.float32)
    alpha = e / den_node
    att = jax.lax.dot_general(
        onehot, alpha * x, (((0,), (0,)), ((), ())),
        preferred_element_type=jnp.float32)
    addp = jax.lax.dot_general(
        onehot, x, (((0,), (0,)), ((), ())),
        preferred_element_type=jnp.float32)

    g = jnp.concatenate([att, addp], axis=1)
    h = jnp.maximum(
        jnp.dot(g, cw1_ref[...], preferred_element_type=jnp.float32)
        + cb1_ref[...], 0.0)
    h = jnp.maximum(
        jnp.dot(h, cw2_ref[...], preferred_element_type=jnp.float32)
        + cb2_ref[...], 0.0)
    out_ref[...] = (
        jnp.dot(h, cw3_ref[...], preferred_element_type=jnp.float32)
        + cb3_ref[...])


def _pool(x, batch, gp, cp):
    return pl.pallas_call(
        _pool_body,
        out_shape=jax.ShapeDtypeStruct((NUM_GRAPHS, 1), jnp.float32),
    )(x, batch.reshape(N_NODES, 1),
      gp['W1'], gp['b1'].reshape(1, -1), gp['W2'], gp['b2'].reshape(1, -1),
      cp['W1'], cp['b1'].reshape(1, -1), cp['W2'], cp['b2'].reshape(1, -1),
      cp['W3'], cp['b3'].reshape(1, -1))


def kernel(x, edge_index, edge_attrs, batch, params):
    src = edge_index[0]
    dst = edge_index[1]
    xi = _node_enc(x, params['node_enc']['W'], params['node_enc']['b'])
    ea = _edge_enc(edge_attrs, params['edge_enc']['W'], params['edge_enc']['b'])
    for i in (1, 2, 3, 4):
        aggr2 = _edge_phase(xi, ea, src, dst)
        xi = _dense_layer(aggr2, xi, params['conv%d' % i], params['ln%d' % i])
    return _pool(xi, batch, params['gate'], params['cls'])
